# baseline (speedup 1.0000x reference)
import functools

import jax
import jax.numpy as jnp
from jax import lax
from jax.experimental import pallas as pl
from jax.experimental.pallas import tpu as pltpu
from jax.experimental.pallas import tpu_sc as plsc

_NUM_FIELDS = 26
_VOCAB = 100000
_EMBED_DIM = 32
_BATCH = 16384

_NC = 2
_NS = 16
_NW = _NC * _NS
_N = _BATCH * _NUM_FIELDS
_ROWS_W = _N // _NW
_CHUNK = 1024
_NCHUNK = _ROWS_W // _CHUNK


_K = 1
_NSTEP = _NCHUNK // _K
_NBUF = 2
_STEP_ROWS = _K * _CHUNK


def _gather_body(idx_hbm, tab_hbm, out_hbm, idx_v, rows_v, sem_g, sem_w):
    wid = lax.axis_index("s") * _NC + lax.axis_index("c")
    pltpu.sync_copy(idx_hbm.at[pl.ds(wid * _NCHUNK, _NCHUNK)], idx_v)

    def fire(step, b):
        for k in range(_K):
            pltpu.async_copy(
                tab_hbm.at[idx_v.at[step * _K + k]],
                rows_v.at[b, pl.ds(k * _CHUNK, _CHUNK)],
                sem_g.at[b],
            )

    fire(0, 0)

    @pl.loop(0, _NSTEP)
    def _step(g):
        b = lax.rem(g, _NBUF)
        nb = lax.rem(g + 1, _NBUF)

        @pl.when(g + 1 < _NSTEP)
        def _prefetch():
            @pl.when(g + 1 >= _NBUF)
            def _reclaim():
                pltpu.make_async_copy(
                    rows_v.at[nb],
                    out_hbm.at[pl.ds(0, _STEP_ROWS)],
                    sem_w.at[nb],
                ).wait()

            fire(g + 1, nb)

        pltpu.make_async_copy(
            tab_hbm.at[pl.ds(0, _STEP_ROWS)], rows_v.at[b], sem_g.at[b]
        ).wait()
        pltpu.async_copy(
            rows_v.at[b],
            out_hbm.at[pl.ds(wid * _ROWS_W + g * _STEP_ROWS, _STEP_ROWS)],
            sem_w.at[b],
        )

    for b in range(_NBUF):
        pltpu.make_async_copy(
            rows_v.at[b], out_hbm.at[pl.ds(0, _STEP_ROWS)], sem_w.at[b]
        ).wait()


_gather = functools.partial(
    pl.kernel,
    out_type=jax.ShapeDtypeStruct((_N, _EMBED_DIM), jnp.float32),
    mesh=plsc.VectorSubcoreMesh(
        core_axis_name="c", subcore_axis_name="s",
        num_cores=_NC, num_subcores=_NS,
    ),
    scratch_types=[
        pltpu.VMEM((_NCHUNK, _CHUNK), jnp.int32),
        pltpu.VMEM((_NBUF, _STEP_ROWS, _EMBED_DIM), jnp.float32),
        pltpu.SemaphoreType.DMA((_NBUF,)),
        pltpu.SemaphoreType.DMA((_NBUF,)),
    ],
    compiler_params=pltpu.CompilerParams(use_tc_tiling_on_sc=False),
)(_gather_body)


def kernel(x, tables):
    offs = jnp.arange(_NUM_FIELDS, dtype=jnp.int32) * _VOCAB
    flat_idx = (x.astype(jnp.int32) + offs[None, :]).reshape(
        _N // _CHUNK, _CHUNK
    )
    tab = tables.reshape(_NUM_FIELDS * _VOCAB, _EMBED_DIM)
    out = _gather(flat_idx, tab)
    return out.reshape(_BATCH, _NUM_FIELDS, _EMBED_DIM)


# speedup vs baseline: 1.3043x; 1.3043x over previous
import functools

import jax
import jax.numpy as jnp
from jax import lax
from jax.experimental import pallas as pl
from jax.experimental.pallas import tpu as pltpu
from jax.experimental.pallas import tpu_sc as plsc

_NUM_FIELDS = 26
_VOCAB = 100000
_EMBED_DIM = 32
_BATCH = 16384

_NC = 2
_NS = 16


def _gather_body(x_hbm, tab_hbm, out_hbm, idx_v, row_v, sem):
    e = lax.axis_index("s") * _NC + lax.axis_index("c")

    @pl.loop(0, _NUM_FIELDS)
    def _field(f):
        pltpu.sync_copy(x_hbm.at[f], idx_v)
        pltpu.async_copy(tab_hbm.at[f, e].at[idx_v], row_v, sem).wait()
        pltpu.sync_copy(row_v, out_hbm.at[f, e])


_gather = functools.partial(
    pl.kernel,
    out_type=jax.ShapeDtypeStruct((_NUM_FIELDS, _EMBED_DIM, _BATCH), jnp.float32),
    mesh=plsc.VectorSubcoreMesh(
        core_axis_name="c", subcore_axis_name="s",
        num_cores=_NC, num_subcores=_NS,
    ),
    scratch_types=[
        pltpu.VMEM((_BATCH,), jnp.int32),
        pltpu.VMEM((_BATCH,), jnp.float32),
        pltpu.SemaphoreType.DMA,
    ],
    compiler_params=pltpu.CompilerParams(use_tc_tiling_on_sc=False),
)(_gather_body)


def kernel(x, tables):
    x_t = x.T.astype(jnp.int32)
    tab_t = jnp.transpose(tables, (0, 2, 1))
    out_t = _gather(x_t, tab_t)
    return jnp.transpose(out_t, (2, 0, 1))
